# baseline (speedup 1.0000x reference)
import jax
import jax.numpy as jnp
import numpy as np
from jax import lax
from jax.experimental import pallas as pl
from jax.experimental.pallas import tpu as pltpu

_B = 16384
_D = 128
_HALF = 64
_BLK = 4096
_NBLK = _B // _BLK
_GROUPS = _BLK // 128

_CC = [0.997372368562427, -19.525529325526072, 60.98837617328467,
       -59.53458698148354]

_SHIFT = np.where(np.arange(_D) < _HALF, 0.0, 0.25).astype(np.float32)[None, :]


def _fused_body(g_ref, h_ref, age_ref, gt_ref, ht_ref, w_ref, shift_ref,
                out_ref):
    age = age_ref[...]
    age = jnp.where(jnp.isnan(age), jnp.zeros_like(age), age)
    ageT = jnp.transpose(age)

    gt = gt_ref[...]
    ht = ht_ref[...]
    gt_hi = gt.astype(jnp.bfloat16)
    ht_hi = ht.astype(jnp.bfloat16)
    gt_lo = (gt - gt_hi.astype(jnp.float32)).astype(jnp.bfloat16)
    ht_lo = (ht - ht_hi.astype(jnp.float32)).astype(jnp.bfloat16)
    z2 = jnp.zeros((2, _D), jnp.bfloat16)
    t16 = jnp.concatenate([gt_hi, ht_hi, z2, gt_lo, ht_lo, z2], axis=0)

    wrow = jnp.transpose(w_ref[...])
    w = jnp.concatenate([wrow, wrow], axis=1)
    shift = shift_ref[...]

    iot = lax.broadcasted_iota(jnp.int32, (8, _D), 0)
    dn = (((0,), (0,)), ((), ()))

    for j in range(_GROUPS):
        g = g_ref[j:j + 1, :]
        h = h_ref[j:j + 1, :]
        oh8 = ((iot == g) | (iot == (h + 3))).astype(jnp.bfloat16)
        oh16 = jnp.concatenate([oh8, oh8], axis=0)
        tab = lax.dot_general(oh16, t16, dn,
                              preferred_element_type=jnp.float32)

        a = ageT[:, j:j + 1]
        u = a * w - shift
        r = u - jnp.round(u)
        x = r * r
        acc = jnp.full(u.shape, _CC[3], jnp.float32)
        acc = acc * x + _CC[2]
        acc = acc * x + _CC[1]
        acc = acc * x + _CC[0]
        out_ref[j * 128:(j + 1) * 128, :] = acc + tab


@jax.jit
def kernel(gender_labels, health_labels, age_values, gender_table,
           health_table, fourier_weight):
    g2 = gender_labels.astype(jnp.int32).reshape(_B // _D, _D)
    h2 = health_labels.astype(jnp.int32).reshape(_B // _D, _D)
    a2 = age_values.reshape(_B // _D, _D)

    grid = (_NBLK,)
    return pl.pallas_call(
        _fused_body,
        grid=grid,
        in_specs=[
            pl.BlockSpec((_GROUPS, _D), lambda i: (i, 0)),
            pl.BlockSpec((_GROUPS, _D), lambda i: (i, 0)),
            pl.BlockSpec((_GROUPS, _D), lambda i: (i, 0)),
            pl.BlockSpec((3, _D), lambda i: (0, 0)),
            pl.BlockSpec((3, _D), lambda i: (0, 0)),
            pl.BlockSpec((_HALF, 1), lambda i: (0, 0)),
            pl.BlockSpec((1, _D), lambda i: (0, 0)),
        ],
        out_specs=pl.BlockSpec((_BLK, _D), lambda i: (i, 0)),
        out_shape=jax.ShapeDtypeStruct((_B, _D), jnp.float32),
        compiler_params=pltpu.CompilerParams(
            dimension_semantics=("parallel",)),
    )(g2, h2, a2, gender_table, health_table, fourier_weight,
      jnp.asarray(_SHIFT))


# speedup vs baseline: 1.0881x; 1.0881x over previous
import jax
import jax.numpy as jnp
import numpy as np
from jax import lax
from jax.experimental import pallas as pl
from jax.experimental.pallas import tpu as pltpu

_B = 16384
_D = 128
_HALF = 64
_BLK = 4096
_NBLK = _B // _BLK
_GROUPS = _BLK // 128

_CC = [0.997372368562427, -19.525529325526072, 60.98837617328467,
       -59.53458698148354]

_SHIFT = np.where(np.arange(_D) < _HALF, 0.0, 0.25).astype(np.float32)[None, :]


def _fused_body(g_ref, h_ref, age_ref, gt_ref, ht_ref, w_ref, shift_ref,
                out_ref):
    f32 = jnp.float32
    bf16 = jnp.bfloat16

    age = age_ref[...]
    age = jnp.where(jnp.isnan(age), jnp.zeros_like(age), age)
    a_hi = age.astype(bf16)
    a_lo = (age - a_hi.astype(f32)).astype(bf16)
    ones = jnp.ones((1, _D), bf16)

    gt = gt_ref[...] + _CC[0] * 0.5
    ht = ht_ref[...] + _CC[0] * 0.5
    gt_hi = gt.astype(bf16)
    ht_hi = ht.astype(bf16)
    gt_lo = (gt - gt_hi.astype(f32)).astype(bf16)
    ht_lo = (ht - ht_hi.astype(f32)).astype(bf16)
    z2 = jnp.zeros((2, _D), bf16)
    t16 = jnp.concatenate([gt_hi, ht_hi, z2, gt_lo, ht_lo, z2], axis=0)

    wrow = jnp.transpose(w_ref[...])
    w = jnp.concatenate([wrow, wrow], axis=1)
    w_hi = w.astype(bf16)
    w_lo = (w - w_hi.astype(f32)).astype(bf16)
    nshift = (-shift_ref[...]).astype(bf16)
    w4 = jnp.concatenate([w_hi, w_hi, w_lo, nshift], axis=0)

    iot = lax.broadcasted_iota(jnp.int32, (8, _D), 0)
    dn = (((0,), (0,)), ((), ()))

    for j in range(_GROUPS):
        g = g_ref[j:j + 1, :]
        h = h_ref[j:j + 1, :]
        oh8 = ((iot == g) | (iot == (h + 3))).astype(bf16)
        oh16 = jnp.concatenate([oh8, oh8], axis=0)
        tab = lax.dot_general(oh16, t16, dn,
                              preferred_element_type=f32)

        lhs4 = jnp.concatenate(
            [a_hi[j:j + 1, :], a_lo[j:j + 1, :], a_hi[j:j + 1, :], ones],
            axis=0)
        u = lax.dot_general(lhs4, w4, dn,
                            preferred_element_type=f32)
        r = u - jnp.round(u)
        x = r * r
        acc = jnp.full(u.shape, _CC[3], f32)
        acc = acc * x + _CC[2]
        acc = acc * x + _CC[1]
        out_ref[j * 128:(j + 1) * 128, :] = acc * x + tab


@jax.jit
def kernel(gender_labels, health_labels, age_values, gender_table,
           health_table, fourier_weight):
    g2 = gender_labels.astype(jnp.int32).reshape(_B // _D, _D)
    h2 = health_labels.astype(jnp.int32).reshape(_B // _D, _D)
    a2 = age_values.reshape(_B // _D, _D)

    grid = (_NBLK,)
    return pl.pallas_call(
        _fused_body,
        grid=grid,
        in_specs=[
            pl.BlockSpec((_GROUPS, _D), lambda i: (i, 0)),
            pl.BlockSpec((_GROUPS, _D), lambda i: (i, 0)),
            pl.BlockSpec((_GROUPS, _D), lambda i: (i, 0)),
            pl.BlockSpec((3, _D), lambda i: (0, 0)),
            pl.BlockSpec((3, _D), lambda i: (0, 0)),
            pl.BlockSpec((_HALF, 1), lambda i: (0, 0)),
            pl.BlockSpec((1, _D), lambda i: (0, 0)),
        ],
        out_specs=pl.BlockSpec((_BLK, _D), lambda i: (i, 0)),
        out_shape=jax.ShapeDtypeStruct((_B, _D), jnp.float32),
        compiler_params=pltpu.CompilerParams(
            dimension_semantics=("parallel",)),
    )(g2, h2, a2, gender_table, health_table, fourier_weight,
      jnp.asarray(_SHIFT))
